# baseline (speedup 1.0000x reference)
import jax
import jax.numpy as jnp
from jax import lax
from jax.experimental import pallas as pl
from jax.experimental.pallas import tpu as pltpu

_NCH = 2


def _som_body(x_hbm, w_hbm, wout_hbm, winner_hbm,
              x_v, w_v, win_v, sem_x0, sem_x1, sem_w, sem_out, sem_win):
    B = x_v.shape[0]
    ch = B // _NCH
    sem_x = (sem_x0, sem_x1)
    cp_w = pltpu.make_async_copy(w_hbm, w_v, sem_w)
    cp_w.start()
    cp_x = [
        pltpu.make_async_copy(x_hbm.at[pl.ds(c * ch, ch)],
                              x_v.at[pl.ds(c * ch, ch)], sem_x[c])
        for c in range(_NCH)
    ]
    for c in range(_NCH):
        cp_x[c].start()
    cp_w.wait()
    cp_out = pltpu.make_async_copy(w_v, wout_hbm, sem_out)
    cp_out.start()
    w = w_v[...]
    wn = jnp.sum(w * w, axis=1, keepdims=True)
    for c in range(_NCH):
        cp_x[c].wait()
        xs = x_v[pl.ds(c * ch, ch), :] * -2.0
        xw = lax.dot_general(xs, w, (((1,), (1,)), ((), ())),
                             preferred_element_type=jnp.float32)
        d2 = xw + wn.T
        win_v[pl.ds(c * ch, ch), :] = (
            jnp.argmin(d2, axis=1).astype(jnp.int32)[:, None])
    cp_win = pltpu.make_async_copy(win_v, winner_hbm, sem_win)
    cp_win.start()
    cp_win.wait()
    cp_out.wait()


def kernel(inputs, W):
    B, D = inputs.shape
    M, _ = W.shape
    wout, _winner = pl.pallas_call(
        _som_body,
        in_specs=[
            pl.BlockSpec(memory_space=pltpu.MemorySpace.HBM),
            pl.BlockSpec(memory_space=pltpu.MemorySpace.HBM),
        ],
        out_specs=[
            pl.BlockSpec(memory_space=pltpu.MemorySpace.HBM),
            pl.BlockSpec(memory_space=pltpu.MemorySpace.HBM),
        ],
        out_shape=(
            jax.ShapeDtypeStruct((M, D), W.dtype),
            jax.ShapeDtypeStruct((B, 1), jnp.int32),
        ),
        scratch_shapes=[
            pltpu.VMEM((B, D), jnp.float32),
            pltpu.VMEM((M, D), jnp.float32),
            pltpu.VMEM((B, 1), jnp.int32),
            pltpu.SemaphoreType.DMA,
            pltpu.SemaphoreType.DMA,
            pltpu.SemaphoreType.DMA,
            pltpu.SemaphoreType.DMA,
            pltpu.SemaphoreType.DMA,
        ],
    )(inputs, W)
    return wout


# speedup vs baseline: 1.0218x; 1.0218x over previous
import jax
import jax.numpy as jnp
from jax import lax
from jax.experimental import pallas as pl
from jax.experimental.pallas import tpu as pltpu


def _som_body(x_hbm, w_hbm, wout_hbm, winner_hbm,
              x_v, w_v, win_v, sem_x, sem_w, sem_out, sem_win):
    cp_x = pltpu.make_async_copy(x_hbm, x_v, sem_x)
    cp_w = pltpu.make_async_copy(w_hbm, w_v, sem_w)
    cp_w.start()
    cp_x.start()
    cp_x.wait()
    xs = x_v[...] * -2.0
    cp_w.wait()
    cp_out = pltpu.make_async_copy(w_v, wout_hbm, sem_out)
    cp_out.start()
    w = w_v[...]
    wn = jnp.sum(w * w, axis=1, keepdims=True)
    xw = lax.dot_general(xs, w, (((1,), (1,)), ((), ())),
                         preferred_element_type=jnp.float32)
    d2 = xw + wn.T
    win_v[...] = jnp.argmin(d2, axis=1).astype(jnp.int32)[:, None]
    cp_win = pltpu.make_async_copy(win_v, winner_hbm, sem_win)
    cp_win.start()
    cp_win.wait()
    cp_out.wait()


def kernel(inputs, W):
    B, D = inputs.shape
    M, _ = W.shape
    wout, _winner = pl.pallas_call(
        _som_body,
        in_specs=[
            pl.BlockSpec(memory_space=pltpu.MemorySpace.HBM),
            pl.BlockSpec(memory_space=pltpu.MemorySpace.HBM),
        ],
        out_specs=[
            pl.BlockSpec(memory_space=pltpu.MemorySpace.HBM),
            pl.BlockSpec(memory_space=pltpu.MemorySpace.HBM),
        ],
        out_shape=(
            jax.ShapeDtypeStruct((M, D), W.dtype),
            jax.ShapeDtypeStruct((B, 1), jnp.int32),
        ),
        scratch_shapes=[
            pltpu.VMEM((B, D), jnp.float32),
            pltpu.VMEM((M, D), jnp.float32),
            pltpu.VMEM((B, 1), jnp.int32),
            pltpu.SemaphoreType.DMA,
            pltpu.SemaphoreType.DMA,
            pltpu.SemaphoreType.DMA,
            pltpu.SemaphoreType.DMA,
        ],
    )(inputs, W)
    return wout


# speedup vs baseline: 1.3793x; 1.3499x over previous
import jax
import jax.numpy as jnp
from jax import lax
from jax.experimental import pallas as pl
from jax.experimental.pallas import tpu as pltpu


def _som_body(x_hbm, w_hbm, wout_hbm, winner_hbm,
              x_v, w_v, win_v, sem_x, sem_w, sem_out, sem_win):
    cp_x = pltpu.make_async_copy(x_hbm, x_v, sem_x)
    cp_w = pltpu.make_async_copy(w_hbm, w_v, sem_w)
    cp_w.start()
    cp_x.start()
    cp_w.wait()
    cp_out = pltpu.make_async_copy(w_v, wout_hbm, sem_out)
    cp_out.start()
    w = w_v[...]
    ws = w * -2.0
    wn = jnp.sum(w * w, axis=1, keepdims=True)
    cp_x.wait()
    x = x_v[...]
    xwt = lax.dot_general(ws, x, (((1,), (1,)), ((), ())),
                          preferred_element_type=jnp.float32)
    d2t = xwt + wn
    win_v[...] = jnp.argmin(d2t, axis=0).astype(jnp.int32)[None, :]
    cp_win = pltpu.make_async_copy(win_v, winner_hbm, sem_win)
    cp_win.start()
    cp_win.wait()
    cp_out.wait()


def kernel(inputs, W):
    B, D = inputs.shape
    M, _ = W.shape
    wout, _winner = pl.pallas_call(
        _som_body,
        in_specs=[
            pl.BlockSpec(memory_space=pltpu.MemorySpace.HBM),
            pl.BlockSpec(memory_space=pltpu.MemorySpace.HBM),
        ],
        out_specs=[
            pl.BlockSpec(memory_space=pltpu.MemorySpace.HBM),
            pl.BlockSpec(memory_space=pltpu.MemorySpace.HBM),
        ],
        out_shape=(
            jax.ShapeDtypeStruct((M, D), W.dtype),
            jax.ShapeDtypeStruct((1, B), jnp.int32),
        ),
        scratch_shapes=[
            pltpu.VMEM((B, D), jnp.float32),
            pltpu.VMEM((M, D), jnp.float32),
            pltpu.VMEM((1, B), jnp.int32),
            pltpu.SemaphoreType.DMA,
            pltpu.SemaphoreType.DMA,
            pltpu.SemaphoreType.DMA,
            pltpu.SemaphoreType.DMA,
        ],
    )(inputs, W)
    return wout
